# Initial kernel scaffold; baseline (speedup 1.0000x reference)
import jax
import jax.numpy as jnp
from jax.experimental import pallas as pl


def kernel(x, edge_index, graph_ids, g_label, W0, b0, W1, b1, Wg1, bg1, Wg2, bg2):
    raise NotImplementedError("write your pallas kernel here")


# speedup vs baseline: 3.8674x; 3.8674x over previous
import math

import jax
import jax.numpy as jnp
from jax import lax
from jax.experimental import pallas as pl
from jax.experimental.pallas import tpu as pltpu
from jax.experimental.pallas import tpu_sc as plsc

N = 10000
E = 320000
G = 64
D = 128

N_PAD = 10240
BLK = 1024
NUM_BLK = N_PAD // BLK
CHUNK = 128
NW = 32
CPW = 79
E_PAD = NW * CHUNK * CPW

BN = 1.0 / math.sqrt(1.0 + 1e-5)
HEAD_W = 128


def _sc_msg_body(h_hbm, src_hbm, dst_hbm, zero_hbm, out_hbm,
                 src_v, dst_v, rows_v, acc_sh, sem):
  cid = lax.axis_index("c")
  sid = lax.axis_index("s")
  rows_per_sub = N_PAD // 16
  rbase = sid * rows_per_sub
  pltpu.sync_copy(zero_hbm.at[pl.ds(rbase, rows_per_sub)],
                  acc_sh.at[pl.ds(rbase, rows_per_sub)])
  plsc.subcore_barrier()

  wid = cid * 16 + sid
  ebase = wid * (CPW * CHUNK)

  def body(j, carry):
    off = ebase + j * CHUNK
    pltpu.sync_copy(src_hbm.at[pl.ds(off, CHUNK)], src_v)
    pltpu.async_copy(h_hbm.at[src_v], rows_v, sem).wait()
    pltpu.sync_copy(dst_hbm.at[pl.ds(off, CHUNK)], dst_v)
    pltpu.sync_copy(rows_v, acc_sh.at[dst_v], add=True)
    return carry

  lax.fori_loop(0, CPW, body, 0)
  plsc.subcore_barrier()
  pltpu.sync_copy(acc_sh.at[pl.ds(rbase, rows_per_sub)],
                  out_hbm.at[cid, pl.ds(rbase, rows_per_sub)])


def _sc_messages(h_pad, src, dst, zeros):
  mesh = plsc.VectorSubcoreMesh(core_axis_name="c", subcore_axis_name="s")
  f = pl.kernel(
      _sc_msg_body,
      out_type=jax.ShapeDtypeStruct((2, N_PAD, D), jnp.float32),
      mesh=mesh,
      scratch_types=[
          pltpu.VMEM((CHUNK,), jnp.int32),
          pltpu.VMEM((CHUNK,), jnp.int32),
          pltpu.VMEM((CHUNK, D), jnp.float32),
          pltpu.VMEM_SHARED((N_PAD, D), jnp.float32),
          pltpu.SemaphoreType.DMA,
      ],
      name="sc_edge_messages",
  )
  return f(h_pad, src, dst, zeros)


def _tc_layer1_body(x_ref, msg_ref, wa_ref, wb_ref, b_ref, o_ref):
  xs = x_ref[...] * BN
  ms = (msg_ref[0] + msg_ref[1]) * BN
  out = jnp.dot(xs, wa_ref[...], preferred_element_type=jnp.float32)
  out += jnp.dot(ms, wb_ref[...], preferred_element_type=jnp.float32)
  out += b_ref[...]
  o_ref[...] = jnp.where(out == 0.0, 1e-18, out)


def _tc_layer1(x_pad, msg, wa, wb, b):
  return pl.pallas_call(
      _tc_layer1_body,
      grid=(NUM_BLK,),
      in_specs=[
          pl.BlockSpec((BLK, D), lambda i: (i, 0)),
          pl.BlockSpec((2, BLK, D), lambda i: (0, i, 0)),
          pl.BlockSpec((D, D), lambda i: (0, 0)),
          pl.BlockSpec((D, D), lambda i: (0, 0)),
          pl.BlockSpec((1, D), lambda i: (0, 0)),
      ],
      out_specs=pl.BlockSpec((BLK, D), lambda i: (i, 0)),
      out_shape=jax.ShapeDtypeStruct((N_PAD, D), jnp.float32),
  )(x_pad, msg, wa, wb, b)


def _tc_layer2_body(h_ref, msg_ref, wa_ref, wb_ref, b_ref, gid_ref,
                    wg1_ref, bg1_ref, wg2_ref, bg2_ref, glab_ref,
                    h_out, ge_out, corr_out):
  i = pl.program_id(0)
  hs = h_ref[...] * BN
  ms = (msg_ref[0] + msg_ref[1]) * BN
  out = jnp.dot(hs, wa_ref[...], preferred_element_type=jnp.float32)
  out += jnp.dot(ms, wb_ref[...], preferred_element_type=jnp.float32)
  out += b_ref[...]
  out = jnp.where(out == 0.0, 1e-18, out)
  h_out[...] = out

  ids = gid_ref[0]
  gi = lax.broadcasted_iota(jnp.int32, (G, BLK), 0)
  onehot = (gi == ids).astype(jnp.float32)
  part = jnp.dot(onehot, out, preferred_element_type=jnp.float32)

  @pl.when(i == 0)
  def _():
    ge_out[...] = jnp.zeros_like(ge_out)

  ge_out[...] += part

  @pl.when(i == pl.num_programs(0) - 1)
  def _():
    ge = ge_out[...]
    z = jnp.dot(ge * BN, wg1_ref[...], preferred_element_type=jnp.float32)
    z += bg1_ref[...]
    z = jnp.maximum(z * BN, 0.0)
    gs = jnp.dot(z, wg2_ref[...], preferred_element_type=jnp.float32)
    gs += bg2_ref[...]
    m = jnp.max(gs, axis=1, keepdims=True)
    col = lax.broadcasted_iota(jnp.int32, (G, HEAD_W), 1)
    pred = jnp.min(jnp.where(gs == m, col, HEAD_W), axis=1, keepdims=True)
    corr_out[0, 0] = jnp.sum((pred == glab_ref[...]).astype(jnp.int32))


def _tc_layer2(h1_pad, msg, wa, wb, b, gid_pad, wg1, bg1, wg2p, bg2p, glab):
  return pl.pallas_call(
      _tc_layer2_body,
      grid=(NUM_BLK,),
      in_specs=[
          pl.BlockSpec((BLK, D), lambda i: (i, 0)),
          pl.BlockSpec((2, BLK, D), lambda i: (0, i, 0)),
          pl.BlockSpec((D, D), lambda i: (0, 0)),
          pl.BlockSpec((D, D), lambda i: (0, 0)),
          pl.BlockSpec((1, D), lambda i: (0, 0)),
          pl.BlockSpec((1, 1, BLK), lambda i: (i, 0, 0)),
          pl.BlockSpec((D, G), lambda i: (0, 0)),
          pl.BlockSpec((1, G), lambda i: (0, 0)),
          pl.BlockSpec((G, HEAD_W), lambda i: (0, 0)),
          pl.BlockSpec((1, HEAD_W), lambda i: (0, 0)),
          pl.BlockSpec((G, 1), lambda i: (0, 0)),
      ],
      out_specs=[
          pl.BlockSpec((BLK, D), lambda i: (i, 0)),
          pl.BlockSpec((G, D), lambda i: (0, 0)),
          pl.BlockSpec(memory_space=pltpu.SMEM),
      ],
      out_shape=[
          jax.ShapeDtypeStruct((N_PAD, D), jnp.float32),
          jax.ShapeDtypeStruct((G, D), jnp.float32),
          jax.ShapeDtypeStruct((1, 1), jnp.int32),
      ],
  )(h1_pad, msg, wa, wb, b, gid_pad, wg1, bg1, wg2p, bg2p, glab)


def kernel(x, edge_index, graph_ids, g_label, W0, b0, W1, b1,
           Wg1, bg1, Wg2, bg2):
  x_pad = jnp.zeros((N_PAD, D), jnp.float32).at[:N].set(x)
  src = jnp.concatenate(
      [edge_index[0], jnp.zeros((E_PAD - E,), edge_index.dtype)])
  dst = jnp.concatenate(
      [edge_index[1], jnp.full((E_PAD - E,), N_PAD - 1, edge_index.dtype)])
  zeros = jnp.zeros((N_PAD, D), jnp.float32)
  gid_pad = jnp.full((N_PAD,), G, graph_ids.dtype).at[:N].set(graph_ids)
  gid_pad = gid_pad.reshape(NUM_BLK, 1, BLK)
  w0a, w0b = W0[:D], W0[D:]
  w1a, w1b = W1[:D], W1[D:]
  b0r = b0.reshape(1, D)
  b1r = b1.reshape(1, D)
  bg1r = bg1.reshape(1, G)
  wg2p = jnp.zeros((G, HEAD_W), jnp.float32).at[:, :10].set(Wg2)
  bg2p = jnp.full((1, HEAD_W), -1e9, jnp.float32).at[0, :10].set(bg2)
  glab = g_label.reshape(G, 1).astype(jnp.int32)

  msg1 = _sc_messages(x_pad, src, dst, zeros)
  h1 = _tc_layer1(x_pad, msg1, w0a, w0b, b0r)

  msg2 = _sc_messages(h1, src, dst, zeros)
  h2, ge, corr = _tc_layer2(h1, msg2, w1a, w1b, b1r, gid_pad,
                            Wg1, bg1r, wg2p, bg2p, glab)

  return (corr[0, 0], G, ge, h2[:N])
